# baseline (speedup 1.0000x reference)
import functools

import jax
import jax.numpy as jnp
from jax import lax
from jax.experimental import pallas as pl
from jax.experimental.pallas import tpu as pltpu
from jax.experimental.pallas import tpu_sc as plsc

_D = 32
_B = 16384

_info = plsc.get_sparse_core_info()
_NC, _NS = _info.num_cores, _info.num_subcores
_NW = _NC * _NS
_BPW = _B // _NW
_CHUNK = 128
_K = _BPW // _CHUNK

_mesh = plsc.VectorSubcoreMesh(core_axis_name="c", subcore_axis_name="s")


@functools.partial(
    pl.kernel,
    mesh=_mesh,
    out_type=jax.ShapeDtypeStruct((_B, _D), jnp.float32),
    scratch_types=[
        pltpu.VMEM((_K, _CHUNK), jnp.int32),
        pltpu.VMEM((_BPW, _D), jnp.float32),
        pltpu.SemaphoreType.DMA,
    ],
    compiler_params=pltpu.CompilerParams(use_tc_tiling_on_sc=False),
)
def _gather_kernel(table_hbm, idx_hbm, out_hbm, idx_v, rows_v, sem):
    wid = lax.axis_index("s") * _NC + lax.axis_index("c")
    pltpu.sync_copy(idx_hbm.at[pl.ds(wid * _K, _K)], idx_v)
    copies = [
        pltpu.async_copy(
            table_hbm.at[idx_v.at[j]],
            rows_v.at[pl.ds(j * _CHUNK, _CHUNK)],
            sem,
        )
        for j in range(_K)
    ]
    for c in copies:
        c.wait()
    pltpu.sync_copy(rows_v, out_hbm.at[pl.ds(wid * _BPW, _BPW)])


def kernel(inputs, table):
    idx = inputs.astype(jnp.int32).reshape(_NW * _K, _CHUNK)
    return _gather_kernel(table, idx)


# speedup vs baseline: 4.1379x; 4.1379x over previous
import functools

import jax
import jax.numpy as jnp
from jax import lax
from jax.experimental import pallas as pl
from jax.experimental.pallas import tpu as pltpu
from jax.experimental.pallas import tpu_sc as plsc

_D = 32
_B = 16384
_L = 16

_info = plsc.get_sparse_core_info()
_NC, _NS = _info.num_cores, _info.num_subcores
_NW = _NC * _NS
_BPW = _B // _NW
_G = _BPW // _L

_mesh = plsc.VectorSubcoreMesh(core_axis_name="c", subcore_axis_name="s")


@functools.partial(
    pl.kernel,
    mesh=_mesh,
    out_type=jax.ShapeDtypeStruct((_D, _B), jnp.float32),
    scratch_types=[
        pltpu.VMEM((_BPW,), jnp.int32),
        pltpu.VMEM((_L, _D, 128), jnp.float32),
        pltpu.VMEM((_D, _BPW), jnp.float32),
        [pltpu.SemaphoreType.DMA] * _L,
    ],
    compiler_params=pltpu.CompilerParams(needs_layout_passes=False),
)
def _gather_kernel(table_t, idx_hbm, out_t, idx_v, ring, slab, sems):
    wid = lax.axis_index("s") * _NC + lax.axis_index("c")
    base = wid * _BPW
    pltpu.sync_copy(idx_hbm.at[pl.ds(base, _BPW)], idx_v)

    rows = lax.iota(jnp.int32, _L)

    def fetch(j, l):
        jt = pl.multiple_of((j // 128) * 128, 128)
        pltpu.async_copy(table_t.at[:, pl.ds(jt, 128)], ring.at[l], sems[l])

    def extract(jr, i, l):
        pltpu.make_async_copy(
            table_t.at[:, pl.ds(0, 128)], ring.at[l], sems[l]
        ).wait()
        col = jnp.full((_L,), jr, jnp.int32)
        dst_col = jnp.full((_L,), i, jnp.int32)
        for h in range(0, _D, _L):
            vals = plsc.load_gather(ring.at[l], [rows + h, col])
            plsc.store_scatter(slab, [rows + h, dst_col], vals)

    vec0 = idx_v[pl.ds(0, _L)]
    for l in range(_L):
        fetch(vec0[l], l)

    def body(g, vec_prev):
        vec = idx_v[pl.ds(g * _L, _L)]
        jr_prev = vec_prev % 128
        for l in range(_L):
            extract(jr_prev[l], (g - 1) * _L + l, l)
            fetch(vec[l], l)
        return vec

    vec_last = lax.fori_loop(1, _G, body, vec0, unroll=False)
    jr_last = vec_last % 128
    for l in range(_L):
        extract(jr_last[l], (_G - 1) * _L + l, l)

    pltpu.sync_copy(slab, out_t.at[:, pl.ds(base, _BPW)])


def kernel(inputs, table):
    out_t = _gather_kernel(table.T, inputs.astype(jnp.int32))
    return out_t.T
